# baseline (speedup 1.0000x reference)
import functools

import jax
import jax.numpy as jnp
from jax import lax
from jax.experimental import pallas as pl
from jax.experimental.pallas import tpu as pltpu
from jax.experimental.pallas import tpu_sc as plsc

_DIM = 64
_LANES = 16
_GRPS = _DIM // _LANES
_NC, _NS = 2, 16
_NW = _NC * _NS
_CHUNK = 100


def _sc_embed_body(nbags_w, nchunks_w, chunks_per_bag, scale,
                   x_hbm, tab_hbm, out_hbm,
                   idx_v, buf0, buf1, outb_v, sem0, sem1):
    wid = lax.axis_index("s") * _NC + lax.axis_index("c")
    row0 = wid * nchunks_w
    pltpu.sync_copy(x_hbm.at[pl.ds(row0, nchunks_w)], idx_v)

    bufs = (buf0, buf1)
    sems = (sem0, sem1)

    pltpu.async_copy(tab_hbm.at[idx_v.at[0]], buf0, sem0)
    pltpu.async_copy(tab_hbm.at[idx_v.at[1]], buf1, sem1)

    @pl.loop(0, nbags_w)
    def _bag(bag):
        accs = [jnp.zeros((_LANES,), jnp.float32) for _ in range(_GRPS)]
        for half in range(chunks_per_bag):
            buf, sem = bufs[half], sems[half]
            jj = bag * chunks_per_bag + half
            pltpu.make_async_copy(tab_hbm.at[idx_v.at[0]], buf, sem).wait()

            def _row(r, a):
                return tuple(a[g] + buf[r, pl.ds(_LANES * g, _LANES)]
                             for g in range(_GRPS))

            accs[:] = lax.fori_loop(0, _CHUNK, _row, tuple(accs), unroll=4)

            @pl.when(jj + chunks_per_bag < nchunks_w)
            def _():
                pltpu.async_copy(tab_hbm.at[idx_v.at[jj + chunks_per_bag]],
                                 buf, sem)

        for g in range(_GRPS):
            outb_v[bag, pl.ds(_LANES * g, _LANES)] = accs[g] * scale

    pltpu.sync_copy(outb_v, out_hbm.at[pl.ds(wid * nbags_w, nbags_w)])


@functools.partial(jax.jit, static_argnums=(2, 3))
def _sc_embed(x_chunks, table, batch, seqlen):
    nchunks = x_chunks.shape[0]
    nbags_w = batch // _NW
    nchunks_w = nchunks // _NW
    chunks_per_bag = nchunks // batch
    mesh = plsc.VectorSubcoreMesh(core_axis_name="c", subcore_axis_name="s",
                                  num_cores=_NC, num_subcores=_NS)
    body = functools.partial(_sc_embed_body, nbags_w, nchunks_w,
                             chunks_per_bag, 1.0 / seqlen)
    return pl.kernel(
        body,
        out_type=jax.ShapeDtypeStruct((batch, _DIM), jnp.float32),
        mesh=mesh,
        compiler_params=pltpu.CompilerParams(use_tc_tiling_on_sc=False),
        scratch_types=[
            pltpu.VMEM((nchunks_w, _CHUNK), jnp.int32),
            pltpu.VMEM((_CHUNK, _DIM), jnp.float32),
            pltpu.VMEM((_CHUNK, _DIM), jnp.float32),
            pltpu.VMEM((nbags_w, _DIM), jnp.float32),
            pltpu.SemaphoreType.DMA,
            pltpu.SemaphoreType.DMA,
        ],
    )(x_chunks, table)


def _tc_head_body(e_ref, w_ref, o_ref):
    logits = lax.dot_general(e_ref[...], w_ref[...],
                             (((1,), (1,)), ((), ())),
                             preferred_element_type=jnp.float32)
    m = jnp.max(logits, axis=-1, keepdims=True)
    l = logits - m
    o_ref[...] = l - jnp.log(jnp.sum(jnp.exp(l), axis=-1, keepdims=True))


def _tc_head(embeds, w):
    batch = embeds.shape[0]
    classes = w.shape[0]
    blk = 1024
    return pl.pallas_call(
        _tc_head_body,
        grid=(batch // blk,),
        in_specs=[
            pl.BlockSpec((blk, _DIM), lambda i: (i, 0)),
            pl.BlockSpec((classes, _DIM), lambda i: (0, 0)),
        ],
        out_specs=pl.BlockSpec((blk, classes), lambda i: (i, 0)),
        out_shape=jax.ShapeDtypeStruct((batch, classes), jnp.float32),
    )(embeds, w)


def kernel(x, A, B):
    batch, seqlen = x.shape
    x_chunks = x.astype(jnp.int32).reshape(batch * seqlen // _CHUNK, _CHUNK)
    embeds = _sc_embed(x_chunks, A, batch, seqlen)
    return _tc_head(embeds, B)


# speedup vs baseline: 1.0687x; 1.0687x over previous
import functools

import jax
import jax.numpy as jnp
from jax import lax
from jax.experimental import pallas as pl
from jax.experimental.pallas import tpu as pltpu
from jax.experimental.pallas import tpu_sc as plsc

_DIM = 64
_LANES = 16
_GRPS = _DIM // _LANES
_NC, _NS = 2, 16
_NW = _NC * _NS
_CHUNK = 100


def _sc_embed_body(nbags_w, nchunks_w, chunks_per_bag, scale,
                   x_hbm, tab_hbm, out_hbm,
                   idx_v, buf0, buf1, outb_v, sem0, sem1):
    wid = lax.axis_index("s") * _NC + lax.axis_index("c")
    row0 = wid * nchunks_w
    pltpu.sync_copy(x_hbm.at[pl.ds(row0, nchunks_w)], idx_v)

    bufs = (buf0, buf1)
    sems = (sem0, sem1)

    pltpu.async_copy(tab_hbm.at[idx_v.at[0]], buf0, sem0)
    pltpu.async_copy(tab_hbm.at[idx_v.at[1]], buf1, sem1)

    @pl.loop(0, nbags_w)
    def _bag(bag):
        accs = [jnp.zeros((_LANES,), jnp.float32) for _ in range(_GRPS)]
        for half in range(chunks_per_bag):
            buf, sem = bufs[half], sems[half]
            jj = bag * chunks_per_bag + half
            pltpu.make_async_copy(tab_hbm.at[idx_v.at[0]], buf, sem).wait()

            def _row(r, a):
                return tuple(a[g] + buf[r, pl.ds(_LANES * g, _LANES)]
                             for g in range(_GRPS))

            accs[:] = lax.fori_loop(0, _CHUNK, _row, tuple(accs), unroll=4)

            @pl.when(jj + chunks_per_bag < nchunks_w)
            def _():
                pltpu.async_copy(tab_hbm.at[idx_v.at[jj + chunks_per_bag]],
                                 buf, sem)

        for g in range(_GRPS):
            outb_v[bag, pl.ds(_LANES * g, _LANES)] = accs[g] * scale

    pltpu.sync_copy(outb_v, out_hbm.at[pl.ds(wid * nbags_w, nbags_w)])


@functools.partial(jax.jit, static_argnums=(2, 3))
def _sc_embed(x_chunks, table, batch, seqlen):
    nchunks = x_chunks.shape[0]
    nbags_w = batch // _NW
    nchunks_w = nchunks // _NW
    chunks_per_bag = nchunks // batch
    mesh = plsc.VectorSubcoreMesh(core_axis_name="c", subcore_axis_name="s",
                                  num_cores=_NC, num_subcores=_NS)
    body = functools.partial(_sc_embed_body, nbags_w, nchunks_w,
                             chunks_per_bag, 1.0 / seqlen)
    return pl.kernel(
        body,
        out_type=jax.ShapeDtypeStruct((batch, _DIM), jnp.float32),
        mesh=mesh,
        compiler_params=pltpu.CompilerParams(use_tc_tiling_on_sc=False),
        scratch_types=[
            pltpu.VMEM((nchunks_w, _CHUNK), jnp.int32),
            pltpu.VMEM((_CHUNK, _DIM), jnp.float32),
            pltpu.VMEM((_CHUNK, _DIM), jnp.float32),
            pltpu.VMEM((nbags_w, _DIM), jnp.float32),
            pltpu.SemaphoreType.DMA,
            pltpu.SemaphoreType.DMA,
        ],
    )(x_chunks, table)


_VB = 2048


def _tc_relayout_body(in_ref, o_ref):
    t = in_ref[...].T
    t3 = t.reshape(_VB // 2, 2, _DIM)
    o_ref[...] = jnp.concatenate([t3[:, 0, :], t3[:, 1, :]], axis=1)


def _tc_relayout(at):
    vocab = at.shape[1]
    grid = (vocab + _VB - 1) // _VB
    return pl.pallas_call(
        _tc_relayout_body,
        grid=(grid,),
        in_specs=[pl.BlockSpec((_DIM, _VB), lambda i: (0, i))],
        out_specs=pl.BlockSpec((_VB // 2, 128), lambda i: (i, 0)),
        out_shape=jax.ShapeDtypeStruct((vocab // 2, 128), jnp.float32),
    )(at)


def _tc_head_body(e_ref, w_ref, o_ref):
    logits = lax.dot_general(e_ref[...], w_ref[...],
                             (((1,), (1,)), ((), ())),
                             preferred_element_type=jnp.float32)
    m = jnp.max(logits, axis=-1, keepdims=True)
    l = logits - m
    o_ref[...] = l - jnp.log(jnp.sum(jnp.exp(l), axis=-1, keepdims=True))


def _tc_head(embeds, w):
    batch = embeds.shape[0]
    classes = w.shape[0]
    blk = 1024
    return pl.pallas_call(
        _tc_head_body,
        grid=(batch // blk,),
        in_specs=[
            pl.BlockSpec((blk, _DIM), lambda i: (i, 0)),
            pl.BlockSpec((classes, _DIM), lambda i: (0, 0)),
        ],
        out_specs=pl.BlockSpec((blk, classes), lambda i: (i, 0)),
        out_shape=jax.ShapeDtypeStruct((batch, classes), jnp.float32),
    )(embeds, w)


def kernel(x, A, B):
    batch, seqlen = x.shape
    x_chunks = x.astype(jnp.int32).reshape(batch * seqlen // _CHUNK, _CHUNK)
    table = _tc_relayout(A.T).reshape(A.shape)
    embeds = _sc_embed(x_chunks, table, batch, seqlen)
    return _tc_head(embeds, B)


# speedup vs baseline: 1.5936x; 1.4912x over previous
import functools

import jax
import jax.numpy as jnp
from jax import lax
from jax.experimental import pallas as pl
from jax.experimental.pallas import tpu as pltpu
from jax.experimental.pallas import tpu_sc as plsc

_DIM = 64
_LANES = 16
_GRPS = _DIM // _LANES
_NC, _NS = 2, 16
_NW = _NC * _NS
_VB = 2048
_CHUNK = 80


def _tc_relayout_body(in1_ref, in2_ref, o_ref):
    stacked = jnp.concatenate([in1_ref[...], in2_ref[...]], axis=0)
    o_ref[...] = stacked.T


def _tc_relayout(at):
    vocab = at.shape[1]
    grid = (vocab + 2 * _VB - 1) // (2 * _VB)
    last = vocab // _VB
    out = pl.pallas_call(
        _tc_relayout_body,
        grid=(grid,),
        in_specs=[
            pl.BlockSpec((_DIM, _VB),
                         lambda i: (0, jnp.minimum(2 * i, last))),
            pl.BlockSpec((_DIM, _VB),
                         lambda i: (0, jnp.minimum(2 * i + 1, last))),
        ],
        out_specs=pl.BlockSpec((_VB, 128), lambda i: (i, 0)),
        out_shape=jax.ShapeDtypeStruct((grid * _VB, 128), jnp.float32),
    )(at, at)
    return out.reshape(grid * 2 * _VB, _DIM)


def _sc_embed_body(nbags_w, nchunks, seqlen,
                   x_hbm, tab_hbm, out_hbm,
                   idxc_v, buf0, buf1, outb_v, sem0, sem1):
    wid = lax.axis_index("s") * _NC + lax.axis_index("c")
    pltpu.sync_copy(x_hbm.at[pl.ds(wid * nchunks, nchunks)], idxc_v)

    bufs = (buf0, buf1)
    sems = (sem0, sem1)
    bpq = 4
    cpq = bpq * seqlen // _CHUNK

    pltpu.async_copy(tab_hbm.at[idxc_v.at[0]], buf0, sem0)
    pltpu.async_copy(tab_hbm.at[idxc_v.at[1]], buf1, sem1)

    def _accum(buf, lo, hi, accs):
        def _row(r, a):
            return tuple(a[g] + buf[r, pl.ds(_LANES * g, _LANES)]
                         for g in range(_GRPS))
        return list(lax.fori_loop(lo, hi, _row, tuple(accs), unroll=4))

    @pl.loop(0, nbags_w // bpq)
    def _quad(quad):
        accs = [[jnp.zeros((_LANES,), jnp.float32) for _ in range(_GRPS)]
                for _ in range(bpq)]
        for c in range(cpq):
            buf, sem = bufs[c % 2], sems[c % 2]
            cid = quad * cpq + c
            pltpu.make_async_copy(tab_hbm.at[idxc_v.at[0]], buf, sem).wait()
            for b in range(bpq):
                lo = max(0, seqlen * b - _CHUNK * c)
                hi = min(_CHUNK, seqlen * (b + 1) - _CHUNK * c)
                if hi > lo:
                    accs[b] = _accum(buf, lo, hi, accs[b])

            @pl.when(cid + 2 < nchunks)
            def _():
                pltpu.async_copy(tab_hbm.at[idxc_v.at[cid + 2]], buf, sem)

        scale = 1.0 / seqlen
        for b in range(bpq):
            for g in range(_GRPS):
                outb_v[bpq * quad + b, pl.ds(_LANES * g, _LANES)] = (
                    accs[b][g] * scale)

    pltpu.sync_copy(outb_v, out_hbm.at[pl.ds(wid * nbags_w, nbags_w)])


@functools.partial(jax.jit, static_argnums=(2, 3))
def _sc_embed(x_chunks, table, batch, seqlen):
    nbags_w = batch // _NW
    nchunks_w = x_chunks.shape[0] // _NW
    mesh = plsc.VectorSubcoreMesh(core_axis_name="c", subcore_axis_name="s",
                                  num_cores=_NC, num_subcores=_NS)
    body = functools.partial(_sc_embed_body, nbags_w, nchunks_w, seqlen)
    return pl.kernel(
        body,
        out_type=jax.ShapeDtypeStruct((batch, _DIM), jnp.float32),
        mesh=mesh,
        compiler_params=pltpu.CompilerParams(use_tc_tiling_on_sc=False),
        scratch_types=[
            pltpu.VMEM((nchunks_w, _CHUNK), jnp.int32),
            pltpu.VMEM((_CHUNK, _DIM), jnp.float32),
            pltpu.VMEM((_CHUNK, _DIM), jnp.float32),
            pltpu.VMEM((nbags_w, _DIM), jnp.float32),
            pltpu.SemaphoreType.DMA,
            pltpu.SemaphoreType.DMA,
        ],
    )(x_chunks, table)


def _tc_head_body(e_ref, w_ref, o_ref):
    logits = lax.dot_general(e_ref[...], w_ref[...],
                             (((1,), (1,)), ((), ())),
                             preferred_element_type=jnp.float32)
    m = jnp.max(logits, axis=-1, keepdims=True)
    l = logits - m
    o_ref[...] = l - jnp.log(jnp.sum(jnp.exp(l), axis=-1, keepdims=True))


def _tc_head(embeds, w):
    batch = embeds.shape[0]
    classes = w.shape[0]
    blk = 1024
    return pl.pallas_call(
        _tc_head_body,
        grid=(batch // blk,),
        in_specs=[
            pl.BlockSpec((blk, _DIM), lambda i: (i, 0)),
            pl.BlockSpec((classes, _DIM), lambda i: (0, 0)),
        ],
        out_specs=pl.BlockSpec((blk, classes), lambda i: (i, 0)),
        out_shape=jax.ShapeDtypeStruct((batch, classes), jnp.float32),
    )(embeds, w)


def _tc_relayout_ident_body(in_ref, o_ref):
    t = in_ref[...].T
    t3 = t.reshape(_VB // 2, 2, _DIM)
    o_ref[...] = jnp.concatenate([t3[:, 0, :], t3[:, 1, :]], axis=1)


def _tc_relayout_ident(at):
    vocab = at.shape[1]
    grid = (vocab + _VB - 1) // _VB
    out = pl.pallas_call(
        _tc_relayout_ident_body,
        grid=(grid,),
        in_specs=[pl.BlockSpec((_DIM, _VB), lambda i: (0, i))],
        out_specs=pl.BlockSpec((_VB // 2, 128), lambda i: (i, 0)),
        out_shape=jax.ShapeDtypeStruct((vocab // 2, 128), jnp.float32),
    )(at)
    return out.reshape(vocab, _DIM)


def kernel(x, A, B):
    batch, seqlen = x.shape
    xi = x.astype(jnp.int32)
    xr = ((xi & jnp.int32(~(2 * _VB - 1)))
          | ((xi & jnp.int32(_VB - 1)) << 1)
          | ((xi >> 11) & jnp.int32(1)))
    x_chunks = xr.reshape(batch * seqlen // _CHUNK, _CHUNK)
    table = _tc_relayout(A.T)
    embeds = _sc_embed(x_chunks, table, batch, seqlen)
    return _tc_head(embeds, B)


# speedup vs baseline: 2.0051x; 1.2582x over previous
import functools

import jax
import jax.numpy as jnp
from jax import lax
from jax.experimental import pallas as pl
from jax.experimental.pallas import tpu as pltpu
from jax.experimental.pallas import tpu_sc as plsc

_DIM = 64
_LANES = 16
_GRPS = _DIM // _LANES
_NC, _NS = 2, 16
_NW = _NC * _NS
_VB = 2048
_CHUNK = 80


def _tc_relayout_body(in1_ref, in2_ref, o_ref):
    stacked = jnp.concatenate([in1_ref[...], in2_ref[...]], axis=0)
    o_ref[...] = stacked.T


def _tc_relayout(at):
    vocab = at.shape[1]
    grid = (vocab + 2 * _VB - 1) // (2 * _VB)
    last = vocab // _VB
    out = pl.pallas_call(
        _tc_relayout_body,
        grid=(grid,),
        in_specs=[
            pl.BlockSpec((_DIM, _VB),
                         lambda i: (0, jnp.minimum(2 * i, last))),
            pl.BlockSpec((_DIM, _VB),
                         lambda i: (0, jnp.minimum(2 * i + 1, last))),
        ],
        out_specs=pl.BlockSpec((_VB, 128), lambda i: (i, 0)),
        out_shape=jax.ShapeDtypeStruct((grid * _VB, 128), jnp.float32),
    )(at, at)
    return out.reshape(grid * 2 * _VB, _DIM)


_NBUF = 5


def _sc_embed_body(nbags_w, nchunks, seqlen,
                   x_hbm, tab_hbm, out_hbm,
                   idxc_v, *rest):
    bufs, sems = rest[:_NBUF], rest[_NBUF + 1:2 * _NBUF + 1]
    outb_v = rest[_NBUF]
    wid = lax.axis_index("s") * _NC + lax.axis_index("c")
    pltpu.sync_copy(x_hbm.at[pl.ds(wid * nchunks, nchunks)], idxc_v)

    bpq = 4
    cpq = bpq * seqlen // _CHUNK

    for k in range(_NBUF):
        pltpu.async_copy(tab_hbm.at[idxc_v.at[k]], bufs[k], sems[k])

    def _accum(buf, lo, hi, accs):
        def _row(r, a):
            return tuple(a[g] + buf[r, pl.ds(_LANES * g, _LANES)]
                         for g in range(_GRPS))
        return list(lax.fori_loop(lo, hi, _row, tuple(accs), unroll=8))

    @pl.loop(0, nbags_w // bpq)
    def _quad(quad):
        accs = [[jnp.zeros((_LANES,), jnp.float32) for _ in range(_GRPS)]
                for _ in range(bpq)]
        for c in range(cpq):
            buf, sem = bufs[c % _NBUF], sems[c % _NBUF]
            cid = quad * cpq + c
            pltpu.make_async_copy(tab_hbm.at[idxc_v.at[0]], buf, sem).wait()
            for b in range(bpq):
                lo = max(0, seqlen * b - _CHUNK * c)
                hi = min(_CHUNK, seqlen * (b + 1) - _CHUNK * c)
                if hi > lo:
                    accs[b] = _accum(buf, lo, hi, accs[b])

            @pl.when(cid + _NBUF < nchunks)
            def _():
                pltpu.async_copy(tab_hbm.at[idxc_v.at[cid + _NBUF]], buf, sem)

        scale = 1.0 / seqlen
        for b in range(bpq):
            for g in range(_GRPS):
                outb_v[bpq * quad + b, pl.ds(_LANES * g, _LANES)] = (
                    accs[b][g] * scale)

    pltpu.sync_copy(outb_v, out_hbm.at[pl.ds(wid * nbags_w, nbags_w)])


@functools.partial(jax.jit, static_argnums=(2, 3))
def _sc_embed(x_chunks, table, batch, seqlen):
    nbags_w = batch // _NW
    nchunks_w = x_chunks.shape[0] // _NW
    mesh = plsc.VectorSubcoreMesh(core_axis_name="c", subcore_axis_name="s",
                                  num_cores=_NC, num_subcores=_NS)
    body = functools.partial(_sc_embed_body, nbags_w, nchunks_w, seqlen)
    return pl.kernel(
        body,
        out_type=jax.ShapeDtypeStruct((batch, _DIM), jnp.float32),
        mesh=mesh,
        compiler_params=pltpu.CompilerParams(use_tc_tiling_on_sc=False),
        scratch_types=(
            [pltpu.VMEM((nchunks_w, _CHUNK), jnp.int32)]
            + [pltpu.VMEM((_CHUNK, _DIM), jnp.float32)] * _NBUF
            + [pltpu.VMEM((nbags_w, _DIM), jnp.float32)]
            + [pltpu.SemaphoreType.DMA] * _NBUF
        ),
    )(x_chunks, table)


def _tc_head_body(e_ref, w_ref, o_ref):
    logits = lax.dot_general(w_ref[...], e_ref[...],
                             (((1,), (1,)), ((), ())),
                             preferred_element_type=jnp.float32)
    m = jnp.max(logits, axis=0, keepdims=True)
    l = logits - m
    o_ref[...] = l - jnp.log(jnp.sum(jnp.exp(l), axis=0, keepdims=True))


def _tc_head(embeds, w):
    batch = embeds.shape[0]
    classes = w.shape[0]
    blk = 1024
    out_t = pl.pallas_call(
        _tc_head_body,
        grid=(batch // blk,),
        in_specs=[
            pl.BlockSpec((blk, _DIM), lambda i: (i, 0)),
            pl.BlockSpec((classes, _DIM), lambda i: (0, 0)),
        ],
        out_specs=pl.BlockSpec((classes, blk), lambda i: (0, i)),
        out_shape=jax.ShapeDtypeStruct((classes, batch), jnp.float32),
    )(embeds, w)
    return out_t.T


def _tc_relayout_ident_body(in_ref, o_ref):
    t = in_ref[...].T
    t3 = t.reshape(_VB // 2, 2, _DIM)
    o_ref[...] = jnp.concatenate([t3[:, 0, :], t3[:, 1, :]], axis=1)


def _tc_relayout_ident(at):
    vocab = at.shape[1]
    grid = (vocab + _VB - 1) // _VB
    out = pl.pallas_call(
        _tc_relayout_ident_body,
        grid=(grid,),
        in_specs=[pl.BlockSpec((_DIM, _VB), lambda i: (0, i))],
        out_specs=pl.BlockSpec((_VB // 2, 128), lambda i: (i, 0)),
        out_shape=jax.ShapeDtypeStruct((vocab // 2, 128), jnp.float32),
    )(at)
    return out.reshape(vocab, _DIM)


def kernel(x, A, B):
    batch, seqlen = x.shape
    xi = x.astype(jnp.int32)
    xr = ((xi & jnp.int32(~(2 * _VB - 1)))
          | ((xi & jnp.int32(_VB - 1)) << 1)
          | ((xi >> 11) & jnp.int32(1)))
    x_chunks = xr.reshape(batch * seqlen // _CHUNK, _CHUNK)
    table = _tc_relayout(A.T)
    embeds = _sc_embed(x_chunks, table, batch, seqlen)
    return _tc_head(embeds, B)


# speedup vs baseline: 2.5564x; 1.2750x over previous
import functools

import jax
import jax.numpy as jnp
from jax import lax
from jax.experimental import pallas as pl
from jax.experimental.pallas import tpu as pltpu
from jax.experimental.pallas import tpu_sc as plsc

_DIM = 64
_LANES = 16
_GRPS = _DIM // _LANES
_NC, _NS = 2, 16
_NW = _NC * _NS
_VB = 4096
_CHUNK = 80


def _tc_relayout_body(in1_ref, in2_ref, o_ref):
    stacked = jnp.concatenate([in1_ref[...], in2_ref[...]], axis=0)
    o_ref[...] = stacked.T


def _tc_relayout(at):
    vocab = at.shape[1]
    grid = (vocab + 2 * _VB - 1) // (2 * _VB)
    last = vocab // _VB
    out = pl.pallas_call(
        _tc_relayout_body,
        grid=(grid,),
        in_specs=[
            pl.BlockSpec((_DIM, _VB),
                         lambda i: (0, jnp.minimum(2 * i, last))),
            pl.BlockSpec((_DIM, _VB),
                         lambda i: (0, jnp.minimum(2 * i + 1, last))),
        ],
        out_specs=pl.BlockSpec((_VB, 128), lambda i: (i, 0)),
        out_shape=jax.ShapeDtypeStruct((grid * _VB, 128), jnp.float32),
    )(at, at)
    return out.reshape(grid * 2 * _VB, _DIM)


_NBUF = 10


def _sc_embed_body(nbags_w, nchunks, seqlen,
                   x_hbm, tab_hbm, out_hbm,
                   idxc_v, *rest):
    bufs, sems = rest[:_NBUF], rest[_NBUF + 1:2 * _NBUF + 1]
    outb_v = rest[_NBUF]
    wid = lax.axis_index("s") * _NC + lax.axis_index("c")
    pltpu.sync_copy(x_hbm.at[pl.ds(wid * nchunks, nchunks)], idxc_v)

    bpq = 4
    cpq = bpq * seqlen // _CHUNK

    for k in range(_NBUF):
        pltpu.async_copy(tab_hbm.at[idxc_v.at[k]], bufs[k], sems[k])

    def _accum(buf, lo, hi, accs):
        def _row(r, a):
            return tuple(a[g] + buf[r, pl.ds(_LANES * g, _LANES)]
                         for g in range(_GRPS))
        return list(lax.fori_loop(lo, hi, _row, tuple(accs), unroll=8))

    @pl.loop(0, nbags_w // bpq)
    def _quad(quad):
        accs = [[jnp.zeros((_LANES,), jnp.float32) for _ in range(_GRPS)]
                for _ in range(bpq)]
        for c in range(cpq):
            buf, sem = bufs[c % _NBUF], sems[c % _NBUF]
            cid = quad * cpq + c
            pltpu.make_async_copy(tab_hbm.at[idxc_v.at[0]], buf, sem).wait()
            for b in range(bpq):
                lo = max(0, seqlen * b - _CHUNK * c)
                hi = min(_CHUNK, seqlen * (b + 1) - _CHUNK * c)
                if hi > lo:
                    accs[b] = _accum(buf, lo, hi, accs[b])

            @pl.when(cid + _NBUF < nchunks)
            def _():
                pltpu.async_copy(tab_hbm.at[idxc_v.at[cid + _NBUF]], buf, sem)

        scale = 1.0 / seqlen
        for b in range(bpq):
            for g in range(_GRPS):
                outb_v[bpq * quad + b, pl.ds(_LANES * g, _LANES)] = (
                    accs[b][g] * scale)

    pltpu.sync_copy(outb_v, out_hbm.at[pl.ds(wid * nbags_w, nbags_w)])


@functools.partial(jax.jit, static_argnums=(2, 3))
def _sc_embed(x_chunks, table, batch, seqlen):
    nbags_w = batch // _NW
    nchunks_w = x_chunks.shape[0] // _NW
    mesh = plsc.VectorSubcoreMesh(core_axis_name="c", subcore_axis_name="s",
                                  num_cores=_NC, num_subcores=_NS)
    body = functools.partial(_sc_embed_body, nbags_w, nchunks_w, seqlen)
    return pl.kernel(
        body,
        out_type=jax.ShapeDtypeStruct((batch, _DIM), jnp.float32),
        mesh=mesh,
        compiler_params=pltpu.CompilerParams(use_tc_tiling_on_sc=False),
        scratch_types=(
            [pltpu.VMEM((nchunks_w, _CHUNK), jnp.int32)]
            + [pltpu.VMEM((_CHUNK, _DIM), jnp.float32)] * _NBUF
            + [pltpu.VMEM((nbags_w, _DIM), jnp.float32)]
            + [pltpu.SemaphoreType.DMA] * _NBUF
        ),
    )(x_chunks, table)


def _tc_head_body(e_ref, w_ref, o_ref):
    logits = lax.dot_general(w_ref[...], e_ref[...],
                             (((1,), (1,)), ((), ())),
                             preferred_element_type=jnp.float32)
    m = jnp.max(logits, axis=0, keepdims=True)
    l = logits - m
    o_ref[...] = l - jnp.log(jnp.sum(jnp.exp(l), axis=0, keepdims=True))


def _tc_head(embeds, w):
    batch = embeds.shape[0]
    classes = w.shape[0]
    blk = 1024
    out_t = pl.pallas_call(
        _tc_head_body,
        grid=(batch // blk,),
        in_specs=[
            pl.BlockSpec((blk, _DIM), lambda i: (i, 0)),
            pl.BlockSpec((classes, _DIM), lambda i: (0, 0)),
        ],
        out_specs=pl.BlockSpec((classes, blk), lambda i: (0, i)),
        out_shape=jax.ShapeDtypeStruct((classes, batch), jnp.float32),
    )(embeds, w)
    return out_t.T


def _tc_relayout_ident_body(in_ref, o_ref):
    t = in_ref[...].T
    t3 = t.reshape(_VB // 2, 2, _DIM)
    o_ref[...] = jnp.concatenate([t3[:, 0, :], t3[:, 1, :]], axis=1)


def _tc_relayout_ident(at):
    vocab = at.shape[1]
    grid = (vocab + _VB - 1) // _VB
    out = pl.pallas_call(
        _tc_relayout_ident_body,
        grid=(grid,),
        in_specs=[pl.BlockSpec((_DIM, _VB), lambda i: (0, i))],
        out_specs=pl.BlockSpec((_VB // 2, 128), lambda i: (i, 0)),
        out_shape=jax.ShapeDtypeStruct((vocab // 2, 128), jnp.float32),
    )(at)
    return out.reshape(vocab, _DIM)


def kernel(x, A, B):
    batch, seqlen = x.shape
    xi = x.astype(jnp.int32)
    xr = ((xi & jnp.int32(~(2 * _VB - 1)))
          | ((xi & jnp.int32(_VB - 1)) << 1)
          | ((xi >> (_VB.bit_length() - 1)) & jnp.int32(1)))
    x_chunks = xr.reshape(batch * seqlen // _CHUNK, _CHUNK)
    table = _tc_relayout(A.T)
    embeds = _sc_embed(x_chunks, table, batch, seqlen)
    return _tc_head(embeds, B)


# speedup vs baseline: 2.8165x; 1.1017x over previous
import functools

import jax
import jax.numpy as jnp
from jax import lax
from jax.experimental import pallas as pl
from jax.experimental.pallas import tpu as pltpu
from jax.experimental.pallas import tpu_sc as plsc

_DIM = 64
_LANES = 16
_GRPS = _DIM // _LANES
_NC, _NS = 2, 16
_NW = _NC * _NS
_VB = 8192
_CHUNK = 80


def _tc_relayout_body(in1_ref, in2_ref, o_ref):
    stacked = jnp.concatenate([in1_ref[...], in2_ref[...]], axis=0)
    o_ref[...] = stacked.T


def _tc_relayout(at):
    vocab = at.shape[1]
    grid = (vocab + 2 * _VB - 1) // (2 * _VB)
    last = vocab // _VB
    out = pl.pallas_call(
        _tc_relayout_body,
        grid=(grid,),
        in_specs=[
            pl.BlockSpec((_DIM, _VB),
                         lambda i: (0, jnp.minimum(2 * i, last))),
            pl.BlockSpec((_DIM, _VB),
                         lambda i: (0, jnp.minimum(2 * i + 1, last))),
        ],
        out_specs=pl.BlockSpec((_VB, 128), lambda i: (i, 0)),
        out_shape=jax.ShapeDtypeStruct((grid * _VB, 128), jnp.float32),
    )(at, at)
    return out.reshape(grid * 2 * _VB, _DIM)


_NBUF = 10


def _sc_embed_body(nbags_w, nchunks, seqlen,
                   x_hbm, tab_hbm, out_hbm,
                   idxc_v, *rest):
    bufs, sems = rest[:_NBUF], rest[_NBUF + 1:2 * _NBUF + 1]
    outb_v = rest[_NBUF]
    wid = lax.axis_index("s") * _NC + lax.axis_index("c")
    pltpu.sync_copy(x_hbm.at[pl.ds(wid * nchunks, nchunks)], idxc_v)

    bpq = 4
    cpq = bpq * seqlen // _CHUNK

    for k in range(_NBUF):
        pltpu.async_copy(tab_hbm.at[idxc_v.at[k]], bufs[k], sems[k])

    def _accum(buf, lo, hi, accs):
        def _row(r, a):
            return tuple(a[g] + buf[r, pl.ds(_LANES * g, _LANES)]
                         for g in range(_GRPS))
        return list(lax.fori_loop(lo, hi, _row, tuple(accs), unroll=8))

    @pl.loop(0, nbags_w // bpq)
    def _quad(quad):
        accs = [[jnp.zeros((_LANES,), jnp.float32) for _ in range(_GRPS)]
                for _ in range(bpq)]
        for c in range(cpq):
            buf, sem = bufs[c % _NBUF], sems[c % _NBUF]
            cid = quad * cpq + c
            pltpu.make_async_copy(tab_hbm.at[idxc_v.at[0]], buf, sem).wait()
            for b in range(bpq):
                lo = max(0, seqlen * b - _CHUNK * c)
                hi = min(_CHUNK, seqlen * (b + 1) - _CHUNK * c)
                if hi > lo:
                    accs[b] = _accum(buf, lo, hi, accs[b])

            @pl.when(cid + _NBUF < nchunks)
            def _():
                pltpu.async_copy(tab_hbm.at[idxc_v.at[cid + _NBUF]], buf, sem)

        scale = 1.0 / seqlen
        for b in range(bpq):
            for g in range(_GRPS):
                outb_v[bpq * quad + b, pl.ds(_LANES * g, _LANES)] = (
                    accs[b][g] * scale)

    pltpu.sync_copy(outb_v, out_hbm.at[pl.ds(wid * nbags_w, nbags_w)])


@functools.partial(jax.jit, static_argnums=(2, 3))
def _sc_embed(x_chunks, table, batch, seqlen):
    nbags_w = batch // _NW
    nchunks_w = x_chunks.shape[0] // _NW
    mesh = plsc.VectorSubcoreMesh(core_axis_name="c", subcore_axis_name="s",
                                  num_cores=_NC, num_subcores=_NS)
    body = functools.partial(_sc_embed_body, nbags_w, nchunks_w, seqlen)
    return pl.kernel(
        body,
        out_type=jax.ShapeDtypeStruct((batch, _DIM), jnp.float32),
        mesh=mesh,
        compiler_params=pltpu.CompilerParams(use_tc_tiling_on_sc=False),
        scratch_types=(
            [pltpu.VMEM((nchunks_w, _CHUNK), jnp.int32)]
            + [pltpu.VMEM((_CHUNK, _DIM), jnp.float32)] * _NBUF
            + [pltpu.VMEM((nbags_w, _DIM), jnp.float32)]
            + [pltpu.SemaphoreType.DMA] * _NBUF
        ),
    )(x_chunks, table)


def _tc_head_body(e_ref, w_ref, o_ref):
    logits = lax.dot_general(w_ref[...], e_ref[...],
                             (((1,), (1,)), ((), ())),
                             preferred_element_type=jnp.float32)
    m = jnp.max(logits, axis=0, keepdims=True)
    l = logits - m
    o_ref[...] = l - jnp.log(jnp.sum(jnp.exp(l), axis=0, keepdims=True))


def _tc_head(embeds, w):
    batch = embeds.shape[0]
    classes = w.shape[0]
    blk = 1024
    out_t = pl.pallas_call(
        _tc_head_body,
        grid=(batch // blk,),
        in_specs=[
            pl.BlockSpec((blk, _DIM), lambda i: (i, 0)),
            pl.BlockSpec((classes, _DIM), lambda i: (0, 0)),
        ],
        out_specs=pl.BlockSpec((classes, blk), lambda i: (0, i)),
        out_shape=jax.ShapeDtypeStruct((classes, batch), jnp.float32),
    )(embeds, w)
    return out_t.T


def _tc_relayout_ident_body(in_ref, o_ref):
    t = in_ref[...].T
    t3 = t.reshape(_VB // 2, 2, _DIM)
    o_ref[...] = jnp.concatenate([t3[:, 0, :], t3[:, 1, :]], axis=1)


def _tc_relayout_ident(at):
    vocab = at.shape[1]
    grid = (vocab + _VB - 1) // _VB
    out = pl.pallas_call(
        _tc_relayout_ident_body,
        grid=(grid,),
        in_specs=[pl.BlockSpec((_DIM, _VB), lambda i: (0, i))],
        out_specs=pl.BlockSpec((_VB // 2, 128), lambda i: (i, 0)),
        out_shape=jax.ShapeDtypeStruct((vocab // 2, 128), jnp.float32),
    )(at)
    return out.reshape(vocab, _DIM)


def kernel(x, A, B):
    batch, seqlen = x.shape
    xi = x.astype(jnp.int32)
    xr = ((xi & jnp.int32(~(2 * _VB - 1)))
          | ((xi & jnp.int32(_VB - 1)) << 1)
          | ((xi >> (_VB.bit_length() - 1)) & jnp.int32(1)))
    x_chunks = xr.reshape(batch * seqlen // _CHUNK, _CHUNK)
    table = _tc_relayout(A.T)
    embeds = _sc_embed(x_chunks, table, batch, seqlen)
    return _tc_head(embeds, B)


# speedup vs baseline: 2.8716x; 1.0196x over previous
import functools

import jax
import jax.numpy as jnp
from jax import lax
from jax.experimental import pallas as pl
from jax.experimental.pallas import tpu as pltpu
from jax.experimental.pallas import tpu_sc as plsc

_DIM = 64
_LANES = 16
_GRPS = _DIM // _LANES
_NC, _NS = 2, 16
_NW = _NC * _NS
_VB = 16384
_CHUNK = 80


def _tc_relayout_body(in1_ref, in2_ref, o_ref):
    stacked = jnp.concatenate([in1_ref[...], in2_ref[...]], axis=0)
    o_ref[...] = stacked.T


def _tc_relayout(at):
    vocab = at.shape[1]
    grid = (vocab + 2 * _VB - 1) // (2 * _VB)
    last = vocab // _VB
    out = pl.pallas_call(
        _tc_relayout_body,
        grid=(grid,),
        in_specs=[
            pl.BlockSpec((_DIM, _VB),
                         lambda i: (0, jnp.minimum(2 * i, last))),
            pl.BlockSpec((_DIM, _VB),
                         lambda i: (0, jnp.minimum(2 * i + 1, last))),
        ],
        out_specs=pl.BlockSpec((_VB, 128), lambda i: (i, 0)),
        out_shape=jax.ShapeDtypeStruct((grid * _VB, 128), jnp.float32),
    )(at, at)
    return out.reshape(grid * 2 * _VB, _DIM)


_NBUF = 10


def _sc_embed_body(nbags_w, nchunks, seqlen,
                   x_hbm, tab_hbm, out_hbm,
                   idxc_v, *rest):
    bufs, sems = rest[:_NBUF], rest[_NBUF + 1:2 * _NBUF + 1]
    outb_v = rest[_NBUF]
    wid = lax.axis_index("s") * _NC + lax.axis_index("c")
    pltpu.sync_copy(x_hbm.at[pl.ds(wid * nchunks, nchunks)], idxc_v)

    bpq = 4
    cpq = bpq * seqlen // _CHUNK

    for k in range(_NBUF):
        pltpu.async_copy(tab_hbm.at[idxc_v.at[k]], bufs[k], sems[k])

    def _accum(buf, lo, hi, accs):
        def _row(r, a):
            return tuple(a[g] + buf[r, pl.ds(_LANES * g, _LANES)]
                         for g in range(_GRPS))
        return list(lax.fori_loop(lo, hi, _row, tuple(accs), unroll=8))

    @pl.loop(0, nbags_w // bpq)
    def _quad(quad):
        accs = [[jnp.zeros((_LANES,), jnp.float32) for _ in range(_GRPS)]
                for _ in range(bpq)]
        for c in range(cpq):
            buf, sem = bufs[c % _NBUF], sems[c % _NBUF]
            cid = quad * cpq + c
            pltpu.make_async_copy(tab_hbm.at[idxc_v.at[0]], buf, sem).wait()
            for b in range(bpq):
                lo = max(0, seqlen * b - _CHUNK * c)
                hi = min(_CHUNK, seqlen * (b + 1) - _CHUNK * c)
                if hi > lo:
                    accs[b] = _accum(buf, lo, hi, accs[b])

            @pl.when(cid + _NBUF < nchunks)
            def _():
                pltpu.async_copy(tab_hbm.at[idxc_v.at[cid + _NBUF]], buf, sem)

        scale = 1.0 / seqlen
        for b in range(bpq):
            for g in range(_GRPS):
                outb_v[bpq * quad + b, pl.ds(_LANES * g, _LANES)] = (
                    accs[b][g] * scale)

    pltpu.sync_copy(outb_v, out_hbm.at[pl.ds(wid * nbags_w, nbags_w)])


@functools.partial(jax.jit, static_argnums=(2, 3))
def _sc_embed(x_chunks, table, batch, seqlen):
    nbags_w = batch // _NW
    nchunks_w = x_chunks.shape[0] // _NW
    mesh = plsc.VectorSubcoreMesh(core_axis_name="c", subcore_axis_name="s",
                                  num_cores=_NC, num_subcores=_NS)
    body = functools.partial(_sc_embed_body, nbags_w, nchunks_w, seqlen)
    return pl.kernel(
        body,
        out_type=jax.ShapeDtypeStruct((batch, _DIM), jnp.float32),
        mesh=mesh,
        compiler_params=pltpu.CompilerParams(use_tc_tiling_on_sc=False),
        scratch_types=(
            [pltpu.VMEM((nchunks_w, _CHUNK), jnp.int32)]
            + [pltpu.VMEM((_CHUNK, _DIM), jnp.float32)] * _NBUF
            + [pltpu.VMEM((nbags_w, _DIM), jnp.float32)]
            + [pltpu.SemaphoreType.DMA] * _NBUF
        ),
    )(x_chunks, table)


def _tc_head_body(e_ref, w_ref, o_ref):
    logits = lax.dot_general(w_ref[...], e_ref[...],
                             (((1,), (1,)), ((), ())),
                             preferred_element_type=jnp.float32)
    m = jnp.max(logits, axis=0, keepdims=True)
    l = logits - m
    o_ref[...] = l - jnp.log(jnp.sum(jnp.exp(l), axis=0, keepdims=True))


def _tc_head(embeds, w):
    batch = embeds.shape[0]
    classes = w.shape[0]
    blk = 1024
    out_t = pl.pallas_call(
        _tc_head_body,
        grid=(batch // blk,),
        in_specs=[
            pl.BlockSpec((blk, _DIM), lambda i: (i, 0)),
            pl.BlockSpec((classes, _DIM), lambda i: (0, 0)),
        ],
        out_specs=pl.BlockSpec((classes, blk), lambda i: (0, i)),
        out_shape=jax.ShapeDtypeStruct((classes, batch), jnp.float32),
    )(embeds, w)
    return out_t.T


def _tc_relayout_ident_body(in_ref, o_ref):
    t = in_ref[...].T
    t3 = t.reshape(_VB // 2, 2, _DIM)
    o_ref[...] = jnp.concatenate([t3[:, 0, :], t3[:, 1, :]], axis=1)


def _tc_relayout_ident(at):
    vocab = at.shape[1]
    grid = (vocab + _VB - 1) // _VB
    out = pl.pallas_call(
        _tc_relayout_ident_body,
        grid=(grid,),
        in_specs=[pl.BlockSpec((_DIM, _VB), lambda i: (0, i))],
        out_specs=pl.BlockSpec((_VB // 2, 128), lambda i: (i, 0)),
        out_shape=jax.ShapeDtypeStruct((vocab // 2, 128), jnp.float32),
    )(at)
    return out.reshape(vocab, _DIM)


def kernel(x, A, B):
    batch, seqlen = x.shape
    xi = x.astype(jnp.int32)
    xr = ((xi & jnp.int32(~(2 * _VB - 1)))
          | ((xi & jnp.int32(_VB - 1)) << 1)
          | ((xi >> (_VB.bit_length() - 1)) & jnp.int32(1)))
    x_chunks = xr.reshape(batch * seqlen // _CHUNK, _CHUNK)
    table = _tc_relayout(A.T)
    embeds = _sc_embed(x_chunks, table, batch, seqlen)
    return _tc_head(embeds, B)


# speedup vs baseline: 3.7360x; 1.3010x over previous
import functools

import jax
import jax.numpy as jnp
from jax import lax
from jax.experimental import pallas as pl
from jax.experimental.pallas import tpu as pltpu
from jax.experimental.pallas import tpu_sc as plsc

_DIM = 64
_LANES = 16
_GRPS = _DIM // _LANES
_NC, _NS = 2, 16
_NW = _NC * _NS
_VB = 8192
_CHUNK = 80
_GPB = 4
_WPR = _DIM // 2


def _bf16_bits(x):
    u = lax.bitcast_convert_type(x, jnp.int32)
    return u + jnp.int32(0x7FFF) + ((u >> 16) & jnp.int32(1))


def _tc_relayout_body(g1_ref, g2_ref, g3_ref, g4_ref, o_ref):
    packed = []
    for g_ref in (g1_ref, g2_ref, g3_ref, g4_ref):
        g = g_ref[...]
        hi = _bf16_bits(g[:_WPR, :]) & jnp.int32(-65536)
        lo = lax.shift_right_logical(_bf16_bits(g[_WPR:, :]), 16)
        packed.append(hi | lo)
    o_ref[...] = jnp.concatenate(packed, axis=0).T


def _tc_relayout(at):
    vocab = at.shape[1]
    grid = (vocab + _GPB * _VB - 1) // (_GPB * _VB)
    last = vocab // _VB
    specs = [
        pl.BlockSpec((_WPR * 2, _VB),
                     functools.partial(
                         lambda k, i: (0, jnp.minimum(_GPB * i + k, last)), k))
        for k in range(_GPB)
    ]
    out = pl.pallas_call(
        _tc_relayout_body,
        grid=(grid,),
        in_specs=specs,
        out_specs=pl.BlockSpec((_VB, 128), lambda i: (i, 0)),
        out_shape=jax.ShapeDtypeStruct((grid * _VB, 128), jnp.int32),
    )(at, at, at, at)
    return out.reshape(grid * _GPB * _VB, _WPR)


_NBUF = 10


def _sc_embed_body(nbags_w, nchunks, seqlen,
                   x_hbm, tab_hbm, out_hbm,
                   idxc_v, *rest):
    bufs, sems = rest[:_NBUF], rest[_NBUF + 1:2 * _NBUF + 1]
    outb_v = rest[_NBUF]
    wid = lax.axis_index("s") * _NC + lax.axis_index("c")
    pltpu.sync_copy(x_hbm.at[pl.ds(wid * nchunks, nchunks)], idxc_v)

    bpq = 4
    cpq = bpq * seqlen // _CHUNK

    for k in range(_NBUF):
        pltpu.async_copy(tab_hbm.at[idxc_v.at[k]], bufs[k], sems[k])

    def _accum(buf, lo, hi, accs):
        def _row(r, a):
            w0 = buf[r, pl.ds(0, _LANES)]
            w1 = buf[r, pl.ds(_LANES, _LANES)]
            bc = functools.partial(lax.bitcast_convert_type,
                                   new_dtype=jnp.float32)
            return (
                a[0] + bc(w0 & jnp.int32(-65536)),
                a[1] + bc(w1 & jnp.int32(-65536)),
                a[2] + bc(w0 << 16),
                a[3] + bc(w1 << 16),
            )
        return list(lax.fori_loop(lo, hi, _row, tuple(accs), unroll=8))

    @pl.loop(0, nbags_w // bpq)
    def _quad(quad):
        accs = [[jnp.zeros((_LANES,), jnp.float32) for _ in range(_GRPS)]
                for _ in range(bpq)]
        for c in range(cpq):
            buf, sem = bufs[c % _NBUF], sems[c % _NBUF]
            cid = quad * cpq + c
            pltpu.make_async_copy(tab_hbm.at[idxc_v.at[0]], buf, sem).wait()
            for b in range(bpq):
                lo = max(0, seqlen * b - _CHUNK * c)
                hi = min(_CHUNK, seqlen * (b + 1) - _CHUNK * c)
                if hi > lo:
                    accs[b] = _accum(buf, lo, hi, accs[b])

            @pl.when(cid + _NBUF < nchunks)
            def _():
                pltpu.async_copy(tab_hbm.at[idxc_v.at[cid + _NBUF]], buf, sem)

        scale = 1.0 / seqlen
        for b in range(bpq):
            for g in range(_GRPS):
                outb_v[bpq * quad + b, pl.ds(_LANES * g, _LANES)] = (
                    accs[b][g] * scale)

    pltpu.sync_copy(outb_v, out_hbm.at[pl.ds(wid * nbags_w, nbags_w)])


@functools.partial(jax.jit, static_argnums=(2, 3))
def _sc_embed(x_chunks, table, batch, seqlen):
    nbags_w = batch // _NW
    nchunks_w = x_chunks.shape[0] // _NW
    mesh = plsc.VectorSubcoreMesh(core_axis_name="c", subcore_axis_name="s",
                                  num_cores=_NC, num_subcores=_NS)
    body = functools.partial(_sc_embed_body, nbags_w, nchunks_w, seqlen)
    return pl.kernel(
        body,
        out_type=jax.ShapeDtypeStruct((batch, _DIM), jnp.float32),
        mesh=mesh,
        compiler_params=pltpu.CompilerParams(use_tc_tiling_on_sc=False),
        scratch_types=(
            [pltpu.VMEM((nchunks_w, _CHUNK), jnp.int32)]
            + [pltpu.VMEM((_CHUNK, _WPR), jnp.int32)] * _NBUF
            + [pltpu.VMEM((nbags_w, _DIM), jnp.float32)]
            + [pltpu.SemaphoreType.DMA] * _NBUF
        ),
    )(x_chunks, table)


def _tc_head_body(e_ref, w_ref, o_ref):
    logits = lax.dot_general(w_ref[...], e_ref[...],
                             (((1,), (1,)), ((), ())),
                             preferred_element_type=jnp.float32)
    m = jnp.max(logits, axis=0, keepdims=True)
    l = logits - m
    o_ref[...] = l - jnp.log(jnp.sum(jnp.exp(l), axis=0, keepdims=True))


def _tc_head(embeds, w):
    batch = embeds.shape[0]
    classes = w.shape[0]
    blk = 1024
    out_t = pl.pallas_call(
        _tc_head_body,
        grid=(batch // blk,),
        in_specs=[
            pl.BlockSpec((blk, _DIM), lambda i: (i, 0)),
            pl.BlockSpec((classes, _DIM), lambda i: (0, 0)),
        ],
        out_specs=pl.BlockSpec((classes, blk), lambda i: (0, i)),
        out_shape=jax.ShapeDtypeStruct((classes, batch), jnp.float32),
    )(embeds, w)
    return out_t.T


def _tc_relayout_ident_body(in_ref, o_ref):
    t = in_ref[...].T
    t3 = t.reshape(_VB // 2, 2, _DIM)
    o_ref[...] = jnp.concatenate([t3[:, 0, :], t3[:, 1, :]], axis=1)


def _tc_relayout_ident(at):
    vocab = at.shape[1]
    grid = (vocab + _VB - 1) // _VB
    out = pl.pallas_call(
        _tc_relayout_ident_body,
        grid=(grid,),
        in_specs=[pl.BlockSpec((_DIM, _VB), lambda i: (0, i))],
        out_specs=pl.BlockSpec((_VB // 2, 128), lambda i: (i, 0)),
        out_shape=jax.ShapeDtypeStruct((vocab // 2, 128), jnp.float32),
    )(at)
    return out.reshape(vocab, _DIM)


def kernel(x, A, B):
    batch, seqlen = x.shape
    xi = x.astype(jnp.int32)
    xr = ((xi & jnp.int32(~(_GPB * _VB - 1)))
          | ((xi & jnp.int32(_VB - 1)) << 2)
          | ((xi >> (_VB.bit_length() - 1)) & jnp.int32(_GPB - 1)))
    x_chunks = xr.reshape(batch * seqlen // _CHUNK, _CHUNK)
    table = _tc_relayout(A.T)
    embeds = _sc_embed(x_chunks, table, batch, seqlen)
    return _tc_head(embeds, B)


# speedup vs baseline: 3.8042x; 1.0183x over previous
import functools

import jax
import jax.numpy as jnp
from jax import lax
from jax.experimental import pallas as pl
from jax.experimental.pallas import tpu as pltpu
from jax.experimental.pallas import tpu_sc as plsc

_DIM = 64
_LANES = 16
_GRPS = _DIM // _LANES
_NC, _NS = 2, 16
_NW = _NC * _NS
_VB = 8192
_CHUNK = 80
_GPB = 4
_WPR = _DIM // 2


def _bf16_bits(x):
    u = lax.bitcast_convert_type(x, jnp.int32)
    return u + jnp.int32(0x7FFF) + ((u >> 16) & jnp.int32(1))


def _tc_relayout_body(g1_ref, g2_ref, g3_ref, g4_ref, o_ref):
    packed = []
    for g_ref in (g1_ref, g2_ref, g3_ref, g4_ref):
        g = g_ref[...]
        hi = _bf16_bits(g[:_WPR, :]) & jnp.int32(-65536)
        lo = lax.shift_right_logical(_bf16_bits(g[_WPR:, :]), 16)
        packed.append(hi | lo)
    o_ref[...] = jnp.concatenate(packed, axis=0).T


def _tc_relayout(at):
    vocab = at.shape[1]
    grid = (vocab + _GPB * _VB - 1) // (_GPB * _VB)
    last = vocab // _VB
    specs = [
        pl.BlockSpec((_WPR * 2, _VB),
                     functools.partial(
                         lambda k, i: (0, jnp.minimum(_GPB * i + k, last)), k))
        for k in range(_GPB)
    ]
    out = pl.pallas_call(
        _tc_relayout_body,
        grid=(grid,),
        in_specs=specs,
        out_specs=pl.BlockSpec((_VB, 128), lambda i: (i, 0)),
        out_shape=jax.ShapeDtypeStruct((grid * _VB, 128), jnp.int32),
    )(at, at, at, at)
    return out.reshape(grid * _GPB * _VB, _WPR)


_NBUF = 10


def _sc_embed_body(nbags_w, nchunks, seqlen,
                   x_hbm, tab_hbm, out_hbm,
                   idxc_v, *rest):
    bufs, sems = rest[:_NBUF], rest[_NBUF + 1:2 * _NBUF + 1]
    outb_v = rest[_NBUF]
    wid = lax.axis_index("s") * _NC + lax.axis_index("c")
    pltpu.sync_copy(x_hbm.at[pl.ds(wid * nchunks, nchunks)], idxc_v)

    bpq = 4
    cpq = bpq * seqlen // _CHUNK

    for k in range(_NBUF):
        pltpu.async_copy(tab_hbm.at[idxc_v.at[k]], bufs[k], sems[k])

    def _accum(buf, lo, hi, accs):
        def _row(r, a):
            w0 = buf[r, pl.ds(0, _LANES)]
            w1 = buf[r, pl.ds(_LANES, _LANES)]
            bc = functools.partial(lax.bitcast_convert_type,
                                   new_dtype=jnp.float32)
            return (
                a[0] + bc(w0),
                a[1] + bc(w1),
                a[2] + bc(w0 << 16),
                a[3] + bc(w1 << 16),
            )
        return list(lax.fori_loop(lo, hi, _row, tuple(accs), unroll=8))

    @pl.loop(0, nbags_w // bpq)
    def _quad(quad):
        accs = [[jnp.zeros((_LANES,), jnp.float32) for _ in range(_GRPS)]
                for _ in range(bpq)]
        for c in range(cpq):
            buf, sem = bufs[c % _NBUF], sems[c % _NBUF]
            cid = quad * cpq + c
            pltpu.make_async_copy(tab_hbm.at[idxc_v.at[0]], buf, sem).wait()
            for b in range(bpq):
                lo = max(0, seqlen * b - _CHUNK * c)
                hi = min(_CHUNK, seqlen * (b + 1) - _CHUNK * c)
                if hi > lo:
                    accs[b] = _accum(buf, lo, hi, accs[b])

            @pl.when(cid + _NBUF < nchunks)
            def _():
                pltpu.async_copy(tab_hbm.at[idxc_v.at[cid + _NBUF]], buf, sem)

        scale = 1.0 / seqlen
        for b in range(bpq):
            for g in range(_GRPS):
                outb_v[bpq * quad + b, pl.ds(_LANES * g, _LANES)] = (
                    accs[b][g] * scale)

    pltpu.sync_copy(outb_v, out_hbm.at[pl.ds(wid * nbags_w, nbags_w)])


@functools.partial(jax.jit, static_argnums=(2, 3))
def _sc_embed(x_chunks, table, batch, seqlen):
    nbags_w = batch // _NW
    nchunks_w = x_chunks.shape[0] // _NW
    mesh = plsc.VectorSubcoreMesh(core_axis_name="c", subcore_axis_name="s",
                                  num_cores=_NC, num_subcores=_NS)
    body = functools.partial(_sc_embed_body, nbags_w, nchunks_w, seqlen)
    return pl.kernel(
        body,
        out_type=jax.ShapeDtypeStruct((batch, _DIM), jnp.float32),
        mesh=mesh,
        compiler_params=pltpu.CompilerParams(use_tc_tiling_on_sc=False),
        scratch_types=(
            [pltpu.VMEM((nchunks_w, _CHUNK), jnp.int32)]
            + [pltpu.VMEM((_CHUNK, _WPR), jnp.int32)] * _NBUF
            + [pltpu.VMEM((nbags_w, _DIM), jnp.float32)]
            + [pltpu.SemaphoreType.DMA] * _NBUF
        ),
    )(x_chunks, table)


def _tc_head_body(e_ref, w_ref, o_ref):
    logits = lax.dot_general(w_ref[...], e_ref[...],
                             (((1,), (1,)), ((), ())),
                             preferred_element_type=jnp.float32)
    m = jnp.max(logits, axis=0, keepdims=True)
    l = logits - m
    o_ref[...] = l - jnp.log(jnp.sum(jnp.exp(l), axis=0, keepdims=True))


def _tc_head(embeds, w):
    batch = embeds.shape[0]
    classes = w.shape[0]
    blk = 2048
    out_t = pl.pallas_call(
        _tc_head_body,
        grid=(batch // blk,),
        in_specs=[
            pl.BlockSpec((blk, _DIM), lambda i: (i, 0)),
            pl.BlockSpec((classes, _DIM), lambda i: (0, 0)),
        ],
        out_specs=pl.BlockSpec((classes, blk), lambda i: (0, i)),
        out_shape=jax.ShapeDtypeStruct((classes, batch), jnp.float32),
    )(embeds, w)
    return out_t.T


def _tc_relayout_ident_body(in_ref, o_ref):
    t = in_ref[...].T
    t3 = t.reshape(_VB // 2, 2, _DIM)
    o_ref[...] = jnp.concatenate([t3[:, 0, :], t3[:, 1, :]], axis=1)


def _tc_relayout_ident(at):
    vocab = at.shape[1]
    grid = (vocab + _VB - 1) // _VB
    out = pl.pallas_call(
        _tc_relayout_ident_body,
        grid=(grid,),
        in_specs=[pl.BlockSpec((_DIM, _VB), lambda i: (0, i))],
        out_specs=pl.BlockSpec((_VB // 2, 128), lambda i: (i, 0)),
        out_shape=jax.ShapeDtypeStruct((vocab // 2, 128), jnp.float32),
    )(at)
    return out.reshape(vocab, _DIM)


def kernel(x, A, B):
    batch, seqlen = x.shape
    xi = x.astype(jnp.int32)
    xr = ((xi & jnp.int32(~(_GPB * _VB - 1)))
          | ((xi & jnp.int32(_VB - 1)) << 2)
          | ((xi >> (_VB.bit_length() - 1)) & jnp.int32(_GPB - 1)))
    x_chunks = xr.reshape(batch * seqlen // _CHUNK, _CHUNK)
    table = _tc_relayout(A.T)
    embeds = _sc_embed(x_chunks, table, batch, seqlen)
    return _tc_head(embeds, B)


# speedup vs baseline: 3.8147x; 1.0027x over previous
import functools

import jax
import jax.numpy as jnp
from jax import lax
from jax.experimental import pallas as pl
from jax.experimental.pallas import tpu as pltpu
from jax.experimental.pallas import tpu_sc as plsc

_DIM = 64
_LANES = 16
_GRPS = _DIM // _LANES
_NC, _NS = 2, 16
_NW = _NC * _NS
_VB = 8192
_CHUNK = 80
_GPB = 4
_WPR = _DIM // 2


def _bf16_bits(x):
    u = lax.bitcast_convert_type(x, jnp.int32)
    return u + jnp.int32(0x7FFF) + ((u >> 16) & jnp.int32(1))


def _tc_relayout_body(g1_ref, g2_ref, g3_ref, g4_ref, o_ref):
    packed = []
    for g_ref in (g1_ref, g2_ref, g3_ref, g4_ref):
        g = g_ref[...]
        hi = _bf16_bits(g[:_WPR, :]) & jnp.int32(-65536)
        lo = lax.shift_right_logical(_bf16_bits(g[_WPR:, :]), 16)
        packed.append(hi | lo)
    o_ref[...] = jnp.concatenate(packed, axis=0).T


def _tc_relayout(at):
    vocab = at.shape[1]
    grid = (vocab + _GPB * _VB - 1) // (_GPB * _VB)
    last = vocab // _VB
    specs = [
        pl.BlockSpec((_WPR * 2, _VB),
                     functools.partial(
                         lambda k, i: (0, jnp.minimum(_GPB * i + k, last)), k))
        for k in range(_GPB)
    ]
    out = pl.pallas_call(
        _tc_relayout_body,
        grid=(grid,),
        in_specs=specs,
        out_specs=pl.BlockSpec((_VB, 128), lambda i: (i, 0)),
        out_shape=jax.ShapeDtypeStruct((grid * _VB, 128), jnp.int32),
    )(at, at, at, at)
    return out.reshape(grid * _GPB * _VB, _WPR)


_NBUF = 10


def _sc_embed_body(nbags_w, nchunks, seqlen,
                   x_hbm, tab_hbm, out_hbm,
                   idxc_v, *rest):
    bufs, sems = rest[:_NBUF], rest[_NBUF + 1:2 * _NBUF + 1]
    outb_v = rest[_NBUF]
    wid = lax.axis_index("s") * _NC + lax.axis_index("c")
    pltpu.sync_copy(x_hbm.at[pl.ds(wid * nchunks, nchunks)], idxc_v)

    bpq = 4
    cpq = bpq * seqlen // _CHUNK

    for k in range(_NBUF):
        pltpu.async_copy(tab_hbm.at[idxc_v.at[k]], bufs[k], sems[k])

    def _accum(buf, lo, hi, accs):
        def _row(r, a):
            w0 = buf[r, pl.ds(0, _LANES)]
            w1 = buf[r, pl.ds(_LANES, _LANES)]
            bc = functools.partial(lax.bitcast_convert_type,
                                   new_dtype=jnp.float32)
            return (
                a[0] + bc(w0),
                a[1] + bc(w1),
                a[2] + bc(w0 << 16),
                a[3] + bc(w1 << 16),
            )
        return list(lax.fori_loop(lo, hi, _row, tuple(accs), unroll=8))

    @pl.loop(0, nbags_w // bpq)
    def _quad(quad):
        accs = [[jnp.zeros((_LANES,), jnp.float32) for _ in range(_GRPS)]
                for _ in range(bpq)]
        for c in range(cpq):
            buf, sem = bufs[c % _NBUF], sems[c % _NBUF]
            cid = quad * cpq + c
            pltpu.make_async_copy(tab_hbm.at[idxc_v.at[0]], buf, sem).wait()
            for b in range(bpq):
                lo = max(0, seqlen * b - _CHUNK * c)
                hi = min(_CHUNK, seqlen * (b + 1) - _CHUNK * c)
                if hi > lo:
                    accs[b] = _accum(buf, lo, hi, accs[b])

            @pl.when(cid + _NBUF < nchunks)
            def _():
                pltpu.async_copy(tab_hbm.at[idxc_v.at[cid + _NBUF]], buf, sem)

        scale = 1.0 / seqlen
        for b in range(bpq):
            for g in range(_GRPS):
                outb_v[bpq * quad + b, pl.ds(_LANES * g, _LANES)] = (
                    accs[b][g] * scale)

    pltpu.sync_copy(outb_v, out_hbm.at[pl.ds(wid * nbags_w, nbags_w)])


@functools.partial(jax.jit, static_argnums=(2, 3))
def _sc_embed(x_chunks, table, batch, seqlen):
    nbags_w = batch // _NW
    nchunks_w = x_chunks.shape[0] // _NW
    mesh = plsc.VectorSubcoreMesh(core_axis_name="c", subcore_axis_name="s",
                                  num_cores=_NC, num_subcores=_NS)
    body = functools.partial(_sc_embed_body, nbags_w, nchunks_w, seqlen)
    return pl.kernel(
        body,
        out_type=jax.ShapeDtypeStruct((batch, _DIM), jnp.float32),
        mesh=mesh,
        compiler_params=pltpu.CompilerParams(use_tc_tiling_on_sc=False),
        scratch_types=(
            [pltpu.VMEM((nchunks_w, _CHUNK), jnp.int32)]
            + [pltpu.VMEM((_CHUNK, _WPR), jnp.int32)] * _NBUF
            + [pltpu.VMEM((nbags_w, _DIM), jnp.float32)]
            + [pltpu.SemaphoreType.DMA] * _NBUF
        ),
    )(x_chunks, table)


def _tc_head_body(e_ref, w_ref, o_ref):
    logits = lax.dot_general(w_ref[...], e_ref[...],
                             (((1,), (1,)), ((), ())),
                             preferred_element_type=jnp.float32)
    m = jnp.max(logits, axis=0, keepdims=True)
    l = logits - m
    o_ref[...] = l - jnp.log(jnp.sum(jnp.exp(l), axis=0, keepdims=True))


def _tc_head(embeds, w):
    batch = embeds.shape[0]
    classes = w.shape[0]
    blk = 2048
    out_t = pl.pallas_call(
        _tc_head_body,
        grid=(batch // blk,),
        in_specs=[
            pl.BlockSpec((blk, _DIM), lambda i: (i, 0)),
            pl.BlockSpec((classes, _DIM), lambda i: (0, 0)),
        ],
        out_specs=pl.BlockSpec((classes, blk), lambda i: (0, i)),
        out_shape=jax.ShapeDtypeStruct((classes, batch), jnp.float32),
    )(embeds, w)
    return out_t.T


def kernel(x, A, B):
    batch, seqlen = x.shape
    xi = x.astype(jnp.int32)
    xr = ((xi & jnp.int32(~(_GPB * _VB - 1)))
          | ((xi & jnp.int32(_VB - 1)) << 2)
          | ((xi >> (_VB.bit_length() - 1)) & jnp.int32(_GPB - 1)))
    x_chunks = xr.reshape(batch * seqlen // _CHUNK, _CHUNK)
    table = _tc_relayout(A.T)
    embeds = _sc_embed(x_chunks, table, batch, seqlen)
    return _tc_head(embeds, B)
